# Initial kernel scaffold; baseline (speedup 1.0000x reference)
import jax
import jax.numpy as jnp
from jax.experimental import pallas as pl


def kernel(x, edge_index, W1, b1, W2, b2):
    raise NotImplementedError("write your pallas kernel here")


# speedup vs baseline: 31.0461x; 31.0461x over previous
import functools

import jax
import jax.numpy as jnp
from jax import lax
from jax.experimental import pallas as pl
from jax.experimental.pallas import tpu as pltpu
from jax.experimental.pallas import tpu_sc as plsc

N_NODES = 10000
N_PAD = 10240
E_PAD = 327680
N_TILES = 32
CHUNKS = 80
CHUNK = 128
ROWS_PER_TILE = N_PAD // 16

_mesh = plsc.VectorSubcoreMesh(core_axis_name="c", subcore_axis_name="s")
_sc_params = pltpu.CompilerParams(use_tc_tiling_on_sc=False)


@functools.partial(
    pl.kernel,
    mesh=_mesh,
    out_type=jax.ShapeDtypeStruct((2, N_PAD), jnp.float32),
    scratch_types=[
        pltpu.VMEM((CHUNKS, CHUNK), jnp.int32),
        pltpu.VMEM((CHUNK,), jnp.float32),
        pltpu.VMEM((ROWS_PER_TILE,), jnp.float32),
        pltpu.VMEM_SHARED((N_PAD,), jnp.float32),
    ],
    compiler_params=_sc_params,
)
def _deg_kernel(dst_hbm, part_hbm, idxd, ones_v, stage, acc):
    c = lax.axis_index("c")
    s = lax.axis_index("s")
    w = c * 16 + s
    pltpu.sync_copy(dst_hbm.at[pl.ds(w * CHUNKS, CHUNKS)], idxd)

    def fill_ones(j, carry):
        ones_v[pl.ds(j * 16, 16)] = jnp.ones((16,), jnp.float32)
        return carry

    lax.fori_loop(0, CHUNK // 16, fill_ones, 0)

    def zero_row(j, carry):
        stage[pl.ds(j * 16, 16)] = jnp.zeros((16,), jnp.float32)
        return carry

    lax.fori_loop(0, ROWS_PER_TILE // 16, zero_row, 0)
    pltpu.sync_copy(stage, acc.at[pl.ds(s * ROWS_PER_TILE, ROWS_PER_TILE)])
    plsc.subcore_barrier()

    def body(ch, carry):
        pltpu.sync_copy(ones_v, acc.at[idxd.at[ch]], add=True)
        return carry

    lax.fori_loop(0, CHUNKS, body, 0)
    plsc.subcore_barrier()
    pltpu.sync_copy(acc.at[pl.ds(s * ROWS_PER_TILE, ROWS_PER_TILE)], stage)
    pltpu.sync_copy(stage, part_hbm.at[c, pl.ds(s * ROWS_PER_TILE, ROWS_PER_TILE)])


@functools.partial(
    pl.kernel,
    mesh=_mesh,
    out_type=jax.ShapeDtypeStruct((2, N_PAD, 16), jnp.float32),
    scratch_types=[
        pltpu.VMEM((CHUNKS, CHUNK), jnp.int32),
        pltpu.VMEM((CHUNKS, CHUNK), jnp.int32),
        pltpu.VMEM((CHUNK, 16), jnp.float32),
        pltpu.VMEM((ROWS_PER_TILE, 16), jnp.float32),
        pltpu.VMEM_SHARED((N_PAD, 16), jnp.float32),
    ],
    compiler_params=_sc_params,
)
def _agg_kernel(src_hbm, dst_hbm, g_hbm, part_hbm, idxs, idxd, rows, stage, acc):
    c = lax.axis_index("c")
    s = lax.axis_index("s")
    w = c * 16 + s
    pltpu.sync_copy(src_hbm.at[pl.ds(w * CHUNKS, CHUNKS)], idxs)
    pltpu.sync_copy(dst_hbm.at[pl.ds(w * CHUNKS, CHUNKS)], idxd)

    def zero_row(j, carry):
        stage[j, :] = jnp.zeros((16,), jnp.float32)
        return carry

    lax.fori_loop(0, ROWS_PER_TILE, zero_row, 0)
    pltpu.sync_copy(stage, acc.at[pl.ds(s * ROWS_PER_TILE, ROWS_PER_TILE)])
    plsc.subcore_barrier()

    def body(ch, carry):
        pltpu.sync_copy(g_hbm.at[idxs.at[ch]], rows)
        pltpu.sync_copy(rows, acc.at[idxd.at[ch]], add=True)
        return carry

    lax.fori_loop(0, CHUNKS, body, 0)
    plsc.subcore_barrier()
    pltpu.sync_copy(acc.at[pl.ds(s * ROWS_PER_TILE, ROWS_PER_TILE)], stage)
    pltpu.sync_copy(stage, part_hbm.at[c, pl.ds(s * ROWS_PER_TILE, ROWS_PER_TILE)])


def _tc1(x_ref, w_ref, degp_ref, g_ref, dis_ref):
    deg = degp_ref[0] + degp_ref[1] + 1.0
    dis = lax.rsqrt(deg)
    h = jnp.dot(x_ref[...], w_ref[...], preferred_element_type=jnp.float32)
    g_ref[...] = h * dis
    dis_ref[...] = dis


def _tc2(p_ref, g1_ref, dis_ref, w2_ref, b1_ref, g2_ref):
    z = dis_ref[...] * (p_ref[0] + p_ref[1] + g1_ref[...]) + b1_ref[...]
    a = jnp.maximum(z, 0.0)
    h2 = jnp.dot(a, w2_ref[...], preferred_element_type=jnp.float32)
    g2_ref[...] = h2 * dis_ref[...]


def _tc3(p_ref, g2_ref, dis_ref, b2_ref, o_ref):
    z = dis_ref[...] * (p_ref[0] + p_ref[1] + g2_ref[...]) + b2_ref[...]
    o_ref[...] = jax.nn.sigmoid(z)


def kernel(x, edge_index, W1, b1, W2, b2):
    f32 = jnp.float32
    ei = edge_index.astype(jnp.int32)
    pad = jnp.full((E_PAD - ei.shape[1],), N_NODES, jnp.int32)
    src2d = jnp.concatenate([ei[0], pad]).reshape(N_TILES * CHUNKS, CHUNK)
    dst2d = jnp.concatenate([ei[1], pad]).reshape(N_TILES * CHUNKS, CHUNK)
    x_pad = jnp.zeros((N_PAD, x.shape[1]), f32).at[:N_NODES].set(x)

    degp = _deg_kernel(dst2d).reshape(2, N_PAD, 1)

    g1, dis = pl.pallas_call(
        _tc1,
        out_shape=[
            jax.ShapeDtypeStruct((N_PAD, 16), f32),
            jax.ShapeDtypeStruct((N_PAD, 1), f32),
        ],
    )(x_pad, W1, degp)

    p1 = _agg_kernel(src2d, dst2d, g1)

    g2 = pl.pallas_call(
        _tc2,
        out_shape=jax.ShapeDtypeStruct((N_PAD, 16), f32),
    )(p1, g1, dis, W2, b1.reshape(1, 16))

    p2 = _agg_kernel(src2d, dst2d, g2)

    out = pl.pallas_call(
        _tc3,
        out_shape=jax.ShapeDtypeStruct((N_PAD, 16), f32),
    )(p2, g2, dis, b2.reshape(1, 16))

    return out[:N_NODES]


# speedup vs baseline: 41.9913x; 1.3525x over previous
import functools

import jax
import jax.numpy as jnp
from jax import lax
from jax.experimental import pallas as pl
from jax.experimental.pallas import tpu as pltpu
from jax.experimental.pallas import tpu_sc as plsc

N_NODES = 10000
N_PAD = 10240
E_PAD = 327680
N_TILES = 32
CHUNKS = 80
CHUNK = 128
ROWS_PER_TILE = N_PAD // 16

_mesh = plsc.VectorSubcoreMesh(core_axis_name="c", subcore_axis_name="s")
_sc_params = pltpu.CompilerParams(use_tc_tiling_on_sc=False)


@functools.partial(
    pl.kernel,
    mesh=_mesh,
    out_type=jax.ShapeDtypeStruct((2, N_PAD), jnp.float32),
    scratch_types=[
        pltpu.VMEM((CHUNKS, CHUNK), jnp.int32),
        pltpu.VMEM((CHUNK,), jnp.float32),
        pltpu.VMEM((ROWS_PER_TILE,), jnp.float32),
        pltpu.VMEM_SHARED((N_PAD,), jnp.float32),
    ],
    compiler_params=_sc_params,
)
def _deg_kernel(dst_hbm, part_hbm, idxd, ones_v, stage, acc):
    c = lax.axis_index("c")
    s = lax.axis_index("s")
    w = c * 16 + s
    pltpu.sync_copy(dst_hbm.at[pl.ds(w * CHUNKS, CHUNKS)], idxd)

    def fill_ones(j, carry):
        ones_v[pl.ds(j * 16, 16)] = jnp.ones((16,), jnp.float32)
        return carry

    lax.fori_loop(0, CHUNK // 16, fill_ones, 0)

    def zero_row(j, carry):
        stage[pl.ds(j * 16, 16)] = jnp.zeros((16,), jnp.float32)
        return carry

    lax.fori_loop(0, ROWS_PER_TILE // 16, zero_row, 0)
    pltpu.sync_copy(stage, acc.at[pl.ds(s * ROWS_PER_TILE, ROWS_PER_TILE)])
    plsc.subcore_barrier()

    def body(ch, carry):
        pltpu.sync_copy(ones_v, acc.at[idxd.at[ch]], add=True)
        return carry

    lax.fori_loop(0, CHUNKS, body, 0)
    plsc.subcore_barrier()
    pltpu.sync_copy(acc.at[pl.ds(s * ROWS_PER_TILE, ROWS_PER_TILE)], stage)
    pltpu.sync_copy(stage, part_hbm.at[c, pl.ds(s * ROWS_PER_TILE, ROWS_PER_TILE)])


@functools.partial(
    pl.kernel,
    mesh=_mesh,
    out_type=jax.ShapeDtypeStruct((2, N_PAD, 16), jnp.float32),
    scratch_types=[
        pltpu.VMEM((CHUNKS, CHUNK), jnp.int32),
        pltpu.VMEM((CHUNKS, CHUNK), jnp.int32),
        pltpu.VMEM((4, CHUNK, 16), jnp.float32),
        pltpu.VMEM((ROWS_PER_TILE, 16), jnp.float32),
        pltpu.VMEM_SHARED((N_PAD, 16), jnp.float32),
        pltpu.SemaphoreType.DMA,
    ],
    compiler_params=_sc_params,
)
def _agg_kernel(src_hbm, dst_hbm, g_hbm, part_hbm, idxs, idxd, rows, stage, acc, sem):
    c = lax.axis_index("c")
    s = lax.axis_index("s")
    w = c * 16 + s
    pltpu.sync_copy(src_hbm.at[pl.ds(w * CHUNKS, CHUNKS)], idxs)
    pltpu.sync_copy(dst_hbm.at[pl.ds(w * CHUNKS, CHUNKS)], idxd)

    def zero_row(j, carry):
        stage[j, :] = jnp.zeros((16,), jnp.float32)
        return carry

    lax.fori_loop(0, ROWS_PER_TILE, zero_row, 0)
    pltpu.sync_copy(stage, acc.at[pl.ds(s * ROWS_PER_TILE, ROWS_PER_TILE)])
    plsc.subcore_barrier()

    for b in range(4):
        pltpu.async_copy(g_hbm.at[idxs.at[b]], rows.at[b], sem)

    def body(ch, carry):
        slot = ch % 4
        pltpu.make_async_copy(g_hbm.at[idxs.at[ch]], rows.at[slot], sem).wait()
        pltpu.sync_copy(rows.at[slot], acc.at[idxd.at[ch]], add=True)

        @pl.when(ch + 4 < CHUNKS)
        def _():
            pltpu.async_copy(g_hbm.at[idxs.at[ch + 4]], rows.at[slot], sem)

        return carry

    lax.fori_loop(0, CHUNKS, body, 0)
    plsc.subcore_barrier()
    pltpu.sync_copy(acc.at[pl.ds(s * ROWS_PER_TILE, ROWS_PER_TILE)], stage)
    pltpu.sync_copy(stage, part_hbm.at[c, pl.ds(s * ROWS_PER_TILE, ROWS_PER_TILE)])


def _tc1(x_ref, w_ref, degp_ref, g_ref, dis_ref):
    deg = degp_ref[0] + degp_ref[1] + 1.0
    dis = lax.rsqrt(deg)
    h = jnp.dot(x_ref[...], w_ref[...], preferred_element_type=jnp.float32)
    g_ref[...] = h * dis
    dis_ref[...] = dis


def _tc2(p_ref, g1_ref, dis_ref, w2_ref, b1_ref, g2_ref):
    z = dis_ref[...] * (p_ref[0] + p_ref[1] + g1_ref[...]) + b1_ref[...]
    a = jnp.maximum(z, 0.0)
    h2 = jnp.dot(a, w2_ref[...], preferred_element_type=jnp.float32)
    g2_ref[...] = h2 * dis_ref[...]


def _tc3(p_ref, g2_ref, dis_ref, b2_ref, o_ref):
    z = dis_ref[...] * (p_ref[0] + p_ref[1] + g2_ref[...]) + b2_ref[...]
    o_ref[...] = jax.nn.sigmoid(z)


def kernel(x, edge_index, W1, b1, W2, b2):
    f32 = jnp.float32
    ei = edge_index.astype(jnp.int32)
    pad = jnp.full((E_PAD - ei.shape[1],), N_NODES, jnp.int32)
    src2d = jnp.concatenate([ei[0], pad]).reshape(N_TILES * CHUNKS, CHUNK)
    dst2d = jnp.concatenate([ei[1], pad]).reshape(N_TILES * CHUNKS, CHUNK)
    x_pad = jnp.zeros((N_PAD, x.shape[1]), f32).at[:N_NODES].set(x)

    degp = _deg_kernel(dst2d).reshape(2, N_PAD, 1)

    g1, dis = pl.pallas_call(
        _tc1,
        out_shape=[
            jax.ShapeDtypeStruct((N_PAD, 16), f32),
            jax.ShapeDtypeStruct((N_PAD, 1), f32),
        ],
    )(x_pad, W1, degp)

    p1 = _agg_kernel(src2d, dst2d, g1)

    g2 = pl.pallas_call(
        _tc2,
        out_shape=jax.ShapeDtypeStruct((N_PAD, 16), f32),
    )(p1, g1, dis, W2, b1.reshape(1, 16))

    p2 = _agg_kernel(src2d, dst2d, g2)

    out = pl.pallas_call(
        _tc3,
        out_shape=jax.ShapeDtypeStruct((N_PAD, 16), f32),
    )(p2, g2, dis, b2.reshape(1, 16))

    return out[:N_NODES]


# speedup vs baseline: 51.8275x; 1.2342x over previous
import functools

import jax
import jax.numpy as jnp
from jax import lax
from jax.experimental import pallas as pl
from jax.experimental.pallas import tpu as pltpu
from jax.experimental.pallas import tpu_sc as plsc

N_NODES = 10000
N_PAD = 10240
N_CHUNKS = 2500
CHUNK = 128
MAX_CHUNKS = 79
ROWS_PER_TILE = N_PAD // 16

_mesh = plsc.VectorSubcoreMesh(core_axis_name="c", subcore_axis_name="s")
_sc_params = pltpu.CompilerParams(use_tc_tiling_on_sc=False)


def _tile_chunks(w):
    n = jnp.where(w < 4, 79, 78)
    off = 78 * w + jnp.minimum(w, 4)
    lo = jnp.where(w < 4, 0, 1)
    return n, off - lo, lo


@functools.partial(
    pl.kernel,
    mesh=_mesh,
    out_type=jax.ShapeDtypeStruct((2, N_PAD), jnp.float32),
    scratch_types=[
        pltpu.VMEM((MAX_CHUNKS, CHUNK), jnp.int32),
        pltpu.VMEM((CHUNK,), jnp.float32),
        pltpu.VMEM((ROWS_PER_TILE,), jnp.float32),
        pltpu.VMEM_SHARED((N_PAD,), jnp.float32),
    ],
    compiler_params=_sc_params,
)
def _deg_kernel(dst_hbm, part_hbm, idxd, ones_v, stage, acc):
    c = lax.axis_index("c")
    s = lax.axis_index("s")
    w = c * 16 + s
    n, win, lo = _tile_chunks(w)
    pltpu.sync_copy(dst_hbm.at[pl.ds(win, MAX_CHUNKS)], idxd)

    def fill_ones(j, carry):
        ones_v[pl.ds(j * 16, 16)] = jnp.ones((16,), jnp.float32)
        return carry

    lax.fori_loop(0, CHUNK // 16, fill_ones, 0)

    def zero_row(j, carry):
        stage[pl.ds(j * 16, 16)] = jnp.zeros((16,), jnp.float32)
        return carry

    lax.fori_loop(0, ROWS_PER_TILE // 16, zero_row, 0)
    pltpu.sync_copy(stage, acc.at[pl.ds(s * ROWS_PER_TILE, ROWS_PER_TILE)])
    plsc.subcore_barrier()

    def body(ch, carry):
        pltpu.sync_copy(ones_v, acc.at[idxd.at[ch]], add=True)
        return carry

    lax.fori_loop(lo, lo + n, body, 0)
    plsc.subcore_barrier()
    pltpu.sync_copy(acc.at[pl.ds(s * ROWS_PER_TILE, ROWS_PER_TILE)], stage)
    pltpu.sync_copy(stage, part_hbm.at[c, pl.ds(s * ROWS_PER_TILE, ROWS_PER_TILE)])


@functools.partial(
    pl.kernel,
    mesh=_mesh,
    out_type=jax.ShapeDtypeStruct((2, N_PAD, 16), jnp.float32),
    scratch_types=[
        pltpu.VMEM((MAX_CHUNKS, CHUNK), jnp.int32),
        pltpu.VMEM((MAX_CHUNKS, CHUNK), jnp.int32),
        pltpu.VMEM((4, CHUNK, 16), jnp.float32),
        pltpu.VMEM((ROWS_PER_TILE, 16), jnp.float32),
        pltpu.VMEM_SHARED((N_PAD, 16), jnp.float32),
        pltpu.SemaphoreType.DMA,
    ],
    compiler_params=_sc_params,
)
def _agg_kernel(src_hbm, dst_hbm, g_hbm, part_hbm, idxs, idxd, rows, stage, acc, sem):
    c = lax.axis_index("c")
    s = lax.axis_index("s")
    w = c * 16 + s
    n, win, lo = _tile_chunks(w)
    pltpu.sync_copy(src_hbm.at[pl.ds(win, MAX_CHUNKS)], idxs)
    pltpu.sync_copy(dst_hbm.at[pl.ds(win, MAX_CHUNKS)], idxd)

    def zero_row(j, carry):
        stage[j, :] = jnp.zeros((16,), jnp.float32)
        return carry

    lax.fori_loop(0, ROWS_PER_TILE, zero_row, 0)
    pltpu.sync_copy(stage, acc.at[pl.ds(s * ROWS_PER_TILE, ROWS_PER_TILE)])
    plsc.subcore_barrier()

    for b in range(4):
        ch0 = lo + b
        pltpu.async_copy(g_hbm.at[idxs.at[ch0]], rows.at[ch0 % 4], sem)

    def body(ch, carry):
        slot = ch % 4
        pltpu.make_async_copy(g_hbm.at[idxs.at[ch]], rows.at[slot], sem).wait()
        pltpu.sync_copy(rows.at[slot], acc.at[idxd.at[ch]], add=True)

        @pl.when(ch + 4 < lo + n)
        def _():
            pltpu.async_copy(g_hbm.at[idxs.at[ch + 4]], rows.at[slot], sem)

        return carry

    lax.fori_loop(lo, lo + n, body, 0)
    plsc.subcore_barrier()
    pltpu.sync_copy(acc.at[pl.ds(s * ROWS_PER_TILE, ROWS_PER_TILE)], stage)
    pltpu.sync_copy(stage, part_hbm.at[c, pl.ds(s * ROWS_PER_TILE, ROWS_PER_TILE)])


def _tc1(x_ref, w_ref, degp_ref, g_ref, dis_ref):
    deg = degp_ref[0] + degp_ref[1] + 1.0
    dis = lax.rsqrt(deg)
    h = jnp.dot(x_ref[...], w_ref[...], preferred_element_type=jnp.float32)
    g_ref[:N_NODES, :] = h * dis[:N_NODES]
    g_ref[N_NODES:, :] = jnp.zeros((N_PAD - N_NODES, 16), jnp.float32)
    dis_ref[...] = dis


def _tc2(p_ref, g1_ref, dis_ref, w2_ref, b1_ref, g2_ref):
    z = dis_ref[...] * (p_ref[0] + p_ref[1] + g1_ref[...]) + b1_ref[...]
    a = jnp.maximum(z, 0.0)
    h2 = jnp.dot(a, w2_ref[...], preferred_element_type=jnp.float32)
    g2_ref[...] = h2 * dis_ref[...]


def _tc3(p_ref, g2_ref, dis_ref, b2_ref, o_ref):
    z = (dis_ref[:N_NODES] * (p_ref[0, :N_NODES] + p_ref[1, :N_NODES]
                              + g2_ref[:N_NODES]) + b2_ref[...])
    o_ref[...] = jax.nn.sigmoid(z)


def kernel(x, edge_index, W1, b1, W2, b2):
    f32 = jnp.float32
    ei = edge_index.astype(jnp.int32)
    src2d = ei[0].reshape(N_CHUNKS, CHUNK)
    dst2d = ei[1].reshape(N_CHUNKS, CHUNK)

    degp = _deg_kernel(dst2d).reshape(2, N_PAD, 1)

    g1, dis = pl.pallas_call(
        _tc1,
        out_shape=[
            jax.ShapeDtypeStruct((N_PAD, 16), f32),
            jax.ShapeDtypeStruct((N_PAD, 1), f32),
        ],
    )(x, W1, degp)

    p1 = _agg_kernel(src2d, dst2d, g1)

    g2 = pl.pallas_call(
        _tc2,
        out_shape=jax.ShapeDtypeStruct((N_PAD, 16), f32),
    )(p1, g1, dis, W2, b1.reshape(1, 16))

    p2 = _agg_kernel(src2d, dst2d, g2)

    return pl.pallas_call(
        _tc3,
        out_shape=jax.ShapeDtypeStruct((N_NODES, 16), f32),
    )(p2, g2, dis, b2.reshape(1, 16))
